# initial kernel scaffold (unmeasured)
import jax
import jax.numpy as jnp
from jax import lax
from jax.experimental import pallas as pl
from jax.experimental.pallas import tpu as pltpu

T = 2048
V_LOCAL = 16384
D = 1024


def kernel(ids, E):
    my_y = lax.axis_index("y")
    offset = my_y * V_LOCAL
    local = ids - offset
    in_range = (local >= 0) & (local < V_LOCAL)
    safe = jnp.where(in_range, local, 0)
    partial = jnp.where(in_range[:, None], E[safe], 0.0).astype(jnp.bfloat16)

    def body(p_ref, out_ref, recv_ref, send_sem, recv_sem):
        x = lax.axis_index("x")
        y = lax.axis_index("y")
        z = lax.axis_index("z")
        partner = (x, 1 - y, z)

        barrier = pltpu.get_barrier_semaphore()
        pl.semaphore_signal(
            barrier, inc=1, device_id=partner,
            device_id_type=pl.DeviceIdType.MESH,
        )
        pl.semaphore_wait(barrier, 1)

        rdma = pltpu.make_async_remote_copy(
            src_ref=p_ref,
            dst_ref=recv_ref,
            send_sem=send_sem,
            recv_sem=recv_sem,
            device_id=partner,
            device_id_type=pl.DeviceIdType.MESH,
        )
        rdma.start()
        rdma.wait()

        out_ref[...] = (
            p_ref[...].astype(jnp.float32) + recv_ref[...].astype(jnp.float32)
        )

    return pl.pallas_call(
        body,
        out_shape=jax.ShapeDtypeStruct((T, D), jnp.float32),
        in_specs=[pl.BlockSpec(memory_space=pltpu.VMEM)],
        out_specs=pl.BlockSpec(memory_space=pltpu.VMEM),
        scratch_shapes=[
            pltpu.VMEM((T, D), jnp.bfloat16),
            pltpu.SemaphoreType.DMA,
            pltpu.SemaphoreType.DMA,
        ],
        compiler_params=pltpu.CompilerParams(collective_id=0),
    )(partial)


# baseline (device time: 97382 ns/iter reference)
import jax
import jax.numpy as jnp
from jax import lax
from jax.experimental import pallas as pl
from jax.experimental.pallas import tpu as pltpu

T = 2048
V_LOCAL = 16384
D = 1024
CR = 256
NC = T // CR


def kernel(ids, E):
    my_y = lax.axis_index("y")
    offset = my_y * V_LOCAL
    local = ids - offset
    in_range = (local >= 0) & (local < V_LOCAL)
    safe = jnp.where(in_range, local, 0).astype(jnp.int32)
    mask = in_range.astype(jnp.float32)[:, None]

    def body(ids_ref, mask_ref, e_ref, out_ref,
             part_ref, recv_ref, stage_ref, gsems, send_sem, recv_sem):
        x = lax.axis_index("x")
        y = lax.axis_index("y")
        z = lax.axis_index("z")
        partner = (x, 1 - y, z)

        barrier = pltpu.get_barrier_semaphore()
        pl.semaphore_signal(
            barrier, inc=1, device_id=partner,
            device_id_type=pl.DeviceIdType.MESH,
        )
        pl.semaphore_wait(barrier, 1)

        def issue_chunk(c, slot):
            def issue(i, carry):
                idx = ids_ref[c * CR + i]
                pltpu.make_async_copy(
                    e_ref.at[idx], stage_ref.at[slot, i], gsems.at[slot]
                ).start()
                return carry
            lax.fori_loop(0, CR, issue, 0, unroll=8)

        def wait_chunk(slot):
            def w(i, carry):
                pltpu.make_async_copy(
                    e_ref.at[0], stage_ref.at[slot, 0], gsems.at[slot]
                ).wait()
                return carry
            lax.fori_loop(0, CR, w, 0, unroll=8)

        def convert_chunk(c, slot):
            rows = pl.ds(c * CR, CR)
            part_ref[rows, :] = (
                stage_ref[slot] * mask_ref[rows, :]
            ).astype(jnp.bfloat16)

        issue_chunk(0, 0)
        for c in range(NC):
            if c + 1 < NC:
                issue_chunk(c + 1, (c + 1) % 2)
            wait_chunk(c % 2)
            convert_chunk(c, c % 2)

        rdma = pltpu.make_async_remote_copy(
            src_ref=part_ref,
            dst_ref=recv_ref,
            send_sem=send_sem,
            recv_sem=recv_sem,
            device_id=partner,
            device_id_type=pl.DeviceIdType.MESH,
        )
        rdma.start()
        rdma.wait()

        out_ref[...] = (
            part_ref[...].astype(jnp.float32) + recv_ref[...].astype(jnp.float32)
        )

    return pl.pallas_call(
        body,
        out_shape=jax.ShapeDtypeStruct((T, D), jnp.float32),
        in_specs=[
            pl.BlockSpec(memory_space=pltpu.SMEM),
            pl.BlockSpec(memory_space=pltpu.VMEM),
            pl.BlockSpec(memory_space=pl.ANY),
        ],
        out_specs=pl.BlockSpec(memory_space=pltpu.VMEM),
        scratch_shapes=[
            pltpu.VMEM((T, D), jnp.bfloat16),
            pltpu.VMEM((T, D), jnp.bfloat16),
            pltpu.VMEM((2, CR, D), jnp.float32),
            pltpu.SemaphoreType.DMA((2,)),
            pltpu.SemaphoreType.DMA,
            pltpu.SemaphoreType.DMA,
        ],
        compiler_params=pltpu.CompilerParams(collective_id=0),
    )(safe, mask, E)


# device time: 80413 ns/iter; 1.2110x vs baseline; 1.2110x over previous
import jax
import jax.numpy as jnp
from jax import lax
from jax.experimental import pallas as pl
from jax.experimental.pallas import tpu as pltpu

T = 2048
V_LOCAL = 16384
D = 1024
CR = 256
NC = T // CR
HC = NC // 2


def kernel(ids, E):
    my_y = lax.axis_index("y")
    offset = my_y * V_LOCAL
    local = ids - offset
    in_range = (local >= 0) & (local < V_LOCAL)
    safe = jnp.where(in_range, local, 0).astype(jnp.int32)
    inr = in_range.astype(jnp.int32)
    mask = in_range.astype(jnp.float32)[:, None]

    def body(ids_ref, inr_ref, mask_ref, e_ref, out_ref,
             part_ref, recv_ref, stage_ref,
             gsems, ysend_sems, yrecv_sems, xsend_sems, xrecv_sems):
        x = lax.axis_index("x")
        y = lax.axis_index("y")
        z = lax.axis_index("z")
        ypartner = (x, 1 - y, z)
        xpartner = (1 - x, y, z)

        barrier = pltpu.get_barrier_semaphore()
        for peer in (ypartner, xpartner):
            pl.semaphore_signal(
                barrier, inc=1, device_id=peer,
                device_id_type=pl.DeviceIdType.MESH,
            )
        pl.semaphore_wait(barrier, 2)

        def gchunk(p):
            if p < HC:
                return x * HC + p
            return (1 - x) * HC + (p - HC)

        def issue_chunk(g, slot):
            base = g * CR

            def issue(i, carry):
                @pl.when(inr_ref[base + i] != 0)
                def _():
                    pltpu.make_async_copy(
                        e_ref.at[ids_ref[base + i]],
                        stage_ref.at[slot, i],
                        gsems.at[slot],
                    ).start()
                return carry

            lax.fori_loop(0, CR, issue, 0, unroll=8)

        def wait_chunk(g, slot):
            base = g * CR

            def w(i, carry):
                @pl.when(inr_ref[base + i] != 0)
                def _():
                    pltpu.make_async_copy(
                        e_ref.at[0],
                        stage_ref.at[slot, 0],
                        gsems.at[slot],
                    ).wait()
                return carry

            lax.fori_loop(0, CR, w, 0, unroll=8)

        def convert_chunk(g, slot):
            rows = pl.ds(g * CR, CR)
            part_ref[rows, :] = stage_ref[slot].astype(jnp.bfloat16)

        def my_rows(c):
            return pl.ds((x * HC + c) * CR, CR)

        def other_rows(c):
            return pl.ds(((1 - x) * HC + c) * CR, CR)

        def combine(rows):
            out_ref[rows, :] = jnp.where(
                mask_ref[rows, :] > 0.0,
                part_ref[rows, :].astype(jnp.float32),
                recv_ref[rows, :].astype(jnp.float32),
            )

        ysends = []
        issue_chunk(gchunk(0), 0)
        for p in range(NC):
            if p + 1 < NC:
                issue_chunk(gchunk(p + 1), (p + 1) % 2)
            wait_chunk(gchunk(p), p % 2)
            convert_chunk(gchunk(p), p % 2)
            if p < HC:
                send = pltpu.make_async_remote_copy(
                    src_ref=part_ref.at[my_rows(p)],
                    dst_ref=recv_ref.at[my_rows(p)],
                    send_sem=ysend_sems.at[p],
                    recv_sem=yrecv_sems.at[p],
                    device_id=ypartner,
                    device_id_type=pl.DeviceIdType.MESH,
                )
                send.start()
                ysends.append(send)

        fwds = []
        for c in range(HC):
            yrecv = pltpu.make_async_remote_copy(
                src_ref=part_ref.at[my_rows(c)],
                dst_ref=recv_ref.at[my_rows(c)],
                send_sem=ysend_sems.at[c],
                recv_sem=yrecv_sems.at[c],
                device_id=ypartner,
                device_id_type=pl.DeviceIdType.MESH,
            )
            yrecv.wait_recv()
            fwd = pltpu.make_async_remote_copy(
                src_ref=recv_ref.at[my_rows(c)],
                dst_ref=recv_ref.at[my_rows(c)],
                send_sem=xsend_sems.at[c],
                recv_sem=xrecv_sems.at[c],
                device_id=xpartner,
                device_id_type=pl.DeviceIdType.MESH,
            )
            fwd.start()
            fwds.append(fwd)
            combine(my_rows(c))

        for c in range(HC):
            xrecv = pltpu.make_async_remote_copy(
                src_ref=recv_ref.at[my_rows(c)],
                dst_ref=recv_ref.at[other_rows(c)],
                send_sem=xsend_sems.at[c],
                recv_sem=xrecv_sems.at[c],
                device_id=xpartner,
                device_id_type=pl.DeviceIdType.MESH,
            )
            xrecv.wait_recv()
            combine(other_rows(c))

        for s in ysends:
            s.wait_send()
        for f in fwds:
            f.wait_send()

    return pl.pallas_call(
        body,
        out_shape=jax.ShapeDtypeStruct((T, D), jnp.float32),
        in_specs=[
            pl.BlockSpec(memory_space=pltpu.SMEM),
            pl.BlockSpec(memory_space=pltpu.SMEM),
            pl.BlockSpec(memory_space=pltpu.VMEM),
            pl.BlockSpec(memory_space=pl.ANY),
        ],
        out_specs=pl.BlockSpec(memory_space=pltpu.VMEM),
        scratch_shapes=[
            pltpu.VMEM((T, D), jnp.bfloat16),
            pltpu.VMEM((T, D), jnp.bfloat16),
            pltpu.VMEM((2, CR, D), jnp.float32),
            pltpu.SemaphoreType.DMA((2,)),
            pltpu.SemaphoreType.DMA((HC,)),
            pltpu.SemaphoreType.DMA((HC,)),
            pltpu.SemaphoreType.DMA((HC,)),
            pltpu.SemaphoreType.DMA((HC,)),
        ],
        compiler_params=pltpu.CompilerParams(collective_id=0),
    )(safe, inr, mask, E)


# device time: 57248 ns/iter; 1.7011x vs baseline; 1.4046x over previous
import jax
import jax.numpy as jnp
from jax import lax
from jax.experimental import pallas as pl
from jax.experimental.pallas import tpu as pltpu

T = 2048
V_LOCAL = 16384
D = 1024
NZ = 4
S = T // NZ
CR = 256


def kernel(ids, E):
    my_y = lax.axis_index("y")
    offset = my_y * V_LOCAL
    local = ids - offset
    in_range = (local >= 0) & (local < V_LOCAL)
    safe = jnp.where(in_range, local, 0).astype(jnp.int32)
    inr = in_range.astype(jnp.int32)
    mask = in_range.astype(jnp.float32)[:, None]

    def body(ids_ref, inr_ref, mask_ref, e_ref, out_ref,
             part_ref, yrecv_ref, res_ref, stage_ref,
             gsems, ysend_sems, yrecv_sems,
             zsend_sems, zrecvL_sems, zrecvR_sems, relR_sems, relL_sems,
             xfwd_sems, xrecvL_sems, xrecvR_sems):
        x = lax.axis_index("x")
        y = lax.axis_index("y")
        z = lax.axis_index("z")
        ypartner = (x, 1 - y, z)
        xpartner = (1 - x, y, z)
        zleft = (x, y, z - 1)
        zright = (x, y, z + 1)
        base = z * S

        barrier = pltpu.get_barrier_semaphore()
        for peer in (ypartner, xpartner):
            pl.semaphore_signal(
                barrier, inc=1, device_id=peer,
                device_id_type=pl.DeviceIdType.MESH,
            )

        @pl.when(z > 0)
        def _():
            pl.semaphore_signal(
                barrier, inc=1, device_id=zleft,
                device_id_type=pl.DeviceIdType.MESH,
            )

        @pl.when(z < NZ - 1)
        def _():
            pl.semaphore_signal(
                barrier, inc=1, device_id=zright,
                device_id_type=pl.DeviceIdType.MESH,
            )

        nwait = 2 + (z > 0).astype(jnp.int32) + (z < NZ - 1).astype(jnp.int32)
        pl.semaphore_wait(barrier, nwait)

        def chunk_of(k):
            return x + k * (1 - 2 * x)

        def issue_chunk(k, slot):
            gbase = base + chunk_of(k) * CR

            def issue(i, carry):
                @pl.when(inr_ref[gbase + i] != 0)
                def _():
                    pltpu.make_async_copy(
                        e_ref.at[ids_ref[gbase + i]],
                        stage_ref.at[slot, i],
                        gsems.at[slot],
                    ).start()
                return carry

            lax.fori_loop(0, CR, issue, 0, unroll=8)

        def wait_chunk(k, slot):
            gbase = base + chunk_of(k) * CR

            def w(i, carry):
                @pl.when(inr_ref[gbase + i] != 0)
                def _():
                    pltpu.make_async_copy(
                        e_ref.at[0],
                        stage_ref.at[slot, 0],
                        gsems.at[slot],
                    ).wait()
                return carry

            lax.fori_loop(0, CR, w, 0, unroll=8)

        ysends = []
        issue_chunk(0, 0)
        issue_chunk(1, 1)
        for k in range(2):
            c = chunk_of(k)
            wait_chunk(k, k)
            part_ref[pl.ds(c * CR, CR), :] = stage_ref[k].astype(jnp.bfloat16)
            send = pltpu.make_async_remote_copy(
                src_ref=part_ref.at[pl.ds(c * CR, CR)],
                dst_ref=yrecv_ref.at[pl.ds(c * CR, CR)],
                send_sem=ysend_sems.at[k],
                recv_sem=yrecv_sems.at[k],
                device_id=ypartner,
                device_id_type=pl.DeviceIdType.MESH,
            )
            send.start()
            ysends.append(send)

        own_rows = pl.ds(base + x * CR, CR)
        zsends = []
        for k in range(2):
            c = chunk_of(k)
            rows_g = pl.ds(base + c * CR, CR)
            yrecv = pltpu.make_async_remote_copy(
                src_ref=part_ref.at[pl.ds(c * CR, CR)],
                dst_ref=yrecv_ref.at[pl.ds(c * CR, CR)],
                send_sem=ysend_sems.at[k],
                recv_sem=yrecv_sems.at[k],
                device_id=ypartner,
                device_id_type=pl.DeviceIdType.MESH,
            )
            yrecv.wait_recv()
            res_ref[rows_g, :] = jnp.where(
                mask_ref[rows_g, :] > 0.0,
                part_ref[pl.ds(c * CR, CR), :],
                yrecv_ref[pl.ds(c * CR, CR), :],
            )
            if k == 0:
                @pl.when(z < NZ - 1)
                def _():
                    s = pltpu.make_async_remote_copy(
                        src_ref=res_ref.at[own_rows],
                        dst_ref=res_ref.at[own_rows],
                        send_sem=zsend_sems.at[0],
                        recv_sem=zrecvL_sems.at[0],
                        device_id=zright,
                        device_id_type=pl.DeviceIdType.MESH,
                    )
                    s.start()

                @pl.when(z > 0)
                def _():
                    s = pltpu.make_async_remote_copy(
                        src_ref=res_ref.at[own_rows],
                        dst_ref=res_ref.at[own_rows],
                        send_sem=zsend_sems.at[1],
                        recv_sem=zrecvR_sems.at[0],
                        device_id=zleft,
                        device_id_type=pl.DeviceIdType.MESH,
                    )
                    s.start()

        def half_rows(origin, xcoord):
            return pl.ds(origin * S + xcoord * CR, CR)

        for d in range(1, NZ):
            oL = z - d
            oR = z + d

            @pl.when(oL >= 0)
            def _(d=d, oL=oL):
                rows = half_rows(oL, x)
                pltpu.make_async_remote_copy(
                    src_ref=res_ref.at[rows],
                    dst_ref=res_ref.at[rows],
                    send_sem=zsend_sems.at[0],
                    recv_sem=zrecvL_sems.at[d - 1],
                    device_id=zleft,
                    device_id_type=pl.DeviceIdType.MESH,
                ).wait_recv()
                @pl.when(z < NZ - 1)
                def _():
                    pltpu.make_async_remote_copy(
                        src_ref=res_ref.at[rows],
                        dst_ref=res_ref.at[rows],
                        send_sem=relR_sems.at[d - 1],
                        recv_sem=zrecvL_sems.at[min(d, NZ - 2)],
                        device_id=zright,
                        device_id_type=pl.DeviceIdType.MESH,
                    ).start()
                pltpu.make_async_remote_copy(
                    src_ref=res_ref.at[rows],
                    dst_ref=res_ref.at[rows],
                    send_sem=xfwd_sems.at[d - 1],
                    recv_sem=xrecvL_sems.at[d - 1],
                    device_id=xpartner,
                    device_id_type=pl.DeviceIdType.MESH,
                ).start()

            @pl.when(oR <= NZ - 1)
            def _(d=d, oR=oR):
                rows = half_rows(oR, x)
                pltpu.make_async_remote_copy(
                    src_ref=res_ref.at[rows],
                    dst_ref=res_ref.at[rows],
                    send_sem=zsend_sems.at[0],
                    recv_sem=zrecvR_sems.at[d - 1],
                    device_id=zright,
                    device_id_type=pl.DeviceIdType.MESH,
                ).wait_recv()
                @pl.when(z > 0)
                def _():
                    pltpu.make_async_remote_copy(
                        src_ref=res_ref.at[rows],
                        dst_ref=res_ref.at[rows],
                        send_sem=relL_sems.at[d - 1],
                        recv_sem=zrecvR_sems.at[min(d, NZ - 2)],
                        device_id=zleft,
                        device_id_type=pl.DeviceIdType.MESH,
                    ).start()
                pltpu.make_async_remote_copy(
                    src_ref=res_ref.at[rows],
                    dst_ref=res_ref.at[rows],
                    send_sem=xfwd_sems.at[3 + d - 1],
                    recv_sem=xrecvR_sems.at[d - 1],
                    device_id=xpartner,
                    device_id_type=pl.DeviceIdType.MESH,
                ).start()

        for d in range(1, NZ):
            oL = z - d
            oR = z + d

            @pl.when(oL >= 0)
            def _(d=d, oL=oL):
                rows = half_rows(oL, 1 - x)
                pltpu.make_async_remote_copy(
                    src_ref=res_ref.at[rows],
                    dst_ref=res_ref.at[rows],
                    send_sem=xfwd_sems.at[d - 1],
                    recv_sem=xrecvL_sems.at[d - 1],
                    device_id=xpartner,
                    device_id_type=pl.DeviceIdType.MESH,
                ).wait_recv()

            @pl.when(oR <= NZ - 1)
            def _(d=d, oR=oR):
                rows = half_rows(oR, 1 - x)
                pltpu.make_async_remote_copy(
                    src_ref=res_ref.at[rows],
                    dst_ref=res_ref.at[rows],
                    send_sem=xfwd_sems.at[3 + d - 1],
                    recv_sem=xrecvR_sems.at[d - 1],
                    device_id=xpartner,
                    device_id_type=pl.DeviceIdType.MESH,
                ).wait_recv()

        out_ref[...] = res_ref[...].astype(jnp.float32)

        for s in ysends:
            s.wait_send()

        @pl.when(z < NZ - 1)
        def _():
            pltpu.make_async_remote_copy(
                src_ref=res_ref.at[own_rows], dst_ref=res_ref.at[own_rows],
                send_sem=zsend_sems.at[0], recv_sem=zrecvL_sems.at[0],
                device_id=zright, device_id_type=pl.DeviceIdType.MESH,
            ).wait_send()

        @pl.when(z > 0)
        def _():
            pltpu.make_async_remote_copy(
                src_ref=res_ref.at[own_rows], dst_ref=res_ref.at[own_rows],
                send_sem=zsend_sems.at[1], recv_sem=zrecvR_sems.at[0],
                device_id=zleft, device_id_type=pl.DeviceIdType.MESH,
            ).wait_send()

        for d in range(1, NZ):
            oL = z - d
            oR = z + d

            @pl.when(oL >= 0)
            def _(d=d, oL=oL):
                rows = half_rows(oL, x)
                @pl.when(z < NZ - 1)
                def _():
                    pltpu.make_async_remote_copy(
                        src_ref=res_ref.at[rows], dst_ref=res_ref.at[rows],
                        send_sem=relR_sems.at[d - 1],
                        recv_sem=zrecvL_sems.at[min(d, NZ - 2)],
                        device_id=zright, device_id_type=pl.DeviceIdType.MESH,
                    ).wait_send()
                pltpu.make_async_remote_copy(
                    src_ref=res_ref.at[rows], dst_ref=res_ref.at[rows],
                    send_sem=xfwd_sems.at[d - 1],
                    recv_sem=xrecvL_sems.at[d - 1],
                    device_id=xpartner, device_id_type=pl.DeviceIdType.MESH,
                ).wait_send()

            @pl.when(oR <= NZ - 1)
            def _(d=d, oR=oR):
                rows = half_rows(oR, x)
                @pl.when(z > 0)
                def _():
                    pltpu.make_async_remote_copy(
                        src_ref=res_ref.at[rows], dst_ref=res_ref.at[rows],
                        send_sem=relL_sems.at[d - 1],
                        recv_sem=zrecvR_sems.at[min(d, NZ - 2)],
                        device_id=zleft, device_id_type=pl.DeviceIdType.MESH,
                    ).wait_send()
                pltpu.make_async_remote_copy(
                    src_ref=res_ref.at[rows], dst_ref=res_ref.at[rows],
                    send_sem=xfwd_sems.at[3 + d - 1],
                    recv_sem=xrecvR_sems.at[d - 1],
                    device_id=xpartner, device_id_type=pl.DeviceIdType.MESH,
                ).wait_send()

    return pl.pallas_call(
        body,
        out_shape=jax.ShapeDtypeStruct((T, D), jnp.float32),
        in_specs=[
            pl.BlockSpec(memory_space=pltpu.SMEM),
            pl.BlockSpec(memory_space=pltpu.SMEM),
            pl.BlockSpec(memory_space=pltpu.VMEM),
            pl.BlockSpec(memory_space=pl.ANY),
        ],
        out_specs=pl.BlockSpec(memory_space=pltpu.VMEM),
        scratch_shapes=[
            pltpu.VMEM((S, D), jnp.bfloat16),
            pltpu.VMEM((S, D), jnp.bfloat16),
            pltpu.VMEM((T, D), jnp.bfloat16),
            pltpu.VMEM((2, CR, D), jnp.float32),
            pltpu.SemaphoreType.DMA((2,)),
            pltpu.SemaphoreType.DMA((2,)),
            pltpu.SemaphoreType.DMA((2,)),
            pltpu.SemaphoreType.DMA((2,)),
            pltpu.SemaphoreType.DMA((NZ - 1,)),
            pltpu.SemaphoreType.DMA((NZ - 1,)),
            pltpu.SemaphoreType.DMA((NZ - 1,)),
            pltpu.SemaphoreType.DMA((NZ - 1,)),
            pltpu.SemaphoreType.DMA((2 * (NZ - 1),)),
            pltpu.SemaphoreType.DMA((NZ - 1,)),
            pltpu.SemaphoreType.DMA((NZ - 1,)),
        ],
        compiler_params=pltpu.CompilerParams(collective_id=0),
    )(safe, inr, mask, E)


# device time: 47603 ns/iter; 2.0457x vs baseline; 1.2026x over previous
import jax
import jax.numpy as jnp
from jax import lax
from jax.experimental import pallas as pl
from jax.experimental.pallas import tpu as pltpu

T = 2048
V_LOCAL = 16384
D = 1024
NZ = 4
S = T // NZ
QR = 128
NQ = S // QR
ND = NZ - 1


def kernel(ids, E):
    my_y = lax.axis_index("y")
    offset = my_y * V_LOCAL
    local = ids - offset
    in_range = (local >= 0) & (local < V_LOCAL)
    safe = jnp.where(in_range, local, 0).astype(jnp.int32)
    inr = in_range.astype(jnp.int32)
    mask = in_range.astype(jnp.float32)[:, None]

    def body(ids_ref, inr_ref, mask_ref, e_ref, out_ref,
             part_ref, yrecv_ref, res_ref, stage_ref,
             gsems, ysend_sems, yrecv_sems,
             zsend_sems, zrecvL_sems, zrecvR_sems, relR_sems, relL_sems,
             xfwdL_sems, xfwdR_sems, xrecvL_sems, xrecvR_sems):
        x = lax.axis_index("x")
        y = lax.axis_index("y")
        z = lax.axis_index("z")
        ypartner = (x, 1 - y, z)
        xpartner = (1 - x, y, z)
        zleft = (x, y, z - 1)
        zright = (x, y, z + 1)
        base = z * S

        barrier = pltpu.get_barrier_semaphore()
        for peer in (ypartner, xpartner):
            pl.semaphore_signal(
                barrier, inc=1, device_id=peer,
                device_id_type=pl.DeviceIdType.MESH,
            )

        @pl.when(z > 0)
        def _():
            pl.semaphore_signal(
                barrier, inc=1, device_id=zleft,
                device_id_type=pl.DeviceIdType.MESH,
            )

        @pl.when(z < NZ - 1)
        def _():
            pl.semaphore_signal(
                barrier, inc=1, device_id=zright,
                device_id_type=pl.DeviceIdType.MESH,
            )

        nwait = 2 + (z > 0).astype(jnp.int32) + (z < NZ - 1).astype(jnp.int32)
        pl.semaphore_wait(barrier, nwait)

        def quarter_of(k):
            if k < 2:
                return x * 2 + k
            return (1 - x) * 2 + (k - 2)

        def issue_chunk(k, slot):
            gbase = base + quarter_of(k) * QR

            def issue(i, carry):
                @pl.when(inr_ref[gbase + i] != 0)
                def _():
                    pltpu.make_async_copy(
                        e_ref.at[ids_ref[gbase + i]],
                        stage_ref.at[slot, i],
                        gsems.at[slot],
                    ).start()
                return carry

            lax.fori_loop(0, QR, issue, 0, unroll=8)

        def wait_chunk(k, slot):
            gbase = base + quarter_of(k) * QR

            def w(i, carry):
                @pl.when(inr_ref[gbase + i] != 0)
                def _():
                    pltpu.make_async_copy(
                        e_ref.at[0],
                        stage_ref.at[slot, 0],
                        gsems.at[slot],
                    ).wait()
                return carry

            lax.fori_loop(0, QR, w, 0, unroll=8)

        def ydesc(k):
            q = quarter_of(k)
            return pltpu.make_async_remote_copy(
                src_ref=part_ref.at[pl.ds(q * QR, QR)],
                dst_ref=yrecv_ref.at[pl.ds(q * QR, QR)],
                send_sem=ysend_sems.at[k],
                recv_sem=yrecv_sems.at[k],
                device_id=ypartner,
                device_id_type=pl.DeviceIdType.MESH,
            )

        def zsub_rows(origin, xcoord, j):
            return pl.ds(origin * S + xcoord * 2 * QR + j * QR, QR)

        def own_send_desc(direction, j):
            rows = zsub_rows(z, x, j)
            return pltpu.make_async_remote_copy(
                src_ref=res_ref.at[rows],
                dst_ref=res_ref.at[rows],
                send_sem=zsend_sems.at[direction * 2 + j],
                recv_sem=(zrecvL_sems if direction == 0 else zrecvR_sems).at[j],
                device_id=zright if direction == 0 else zleft,
                device_id_type=pl.DeviceIdType.MESH,
            )

        def combine(k):
            q = quarter_of(k)
            rows_g = pl.ds(base + q * QR, QR)
            ydesc(k).wait_recv()
            res_ref[rows_g, :] = jnp.where(
                mask_ref[rows_g, :] > 0.0,
                part_ref[pl.ds(q * QR, QR), :],
                yrecv_ref[pl.ds(q * QR, QR), :],
            )
            out_ref[rows_g, :] = res_ref[rows_g, :].astype(jnp.float32)
            if k < 2:
                @pl.when(z < NZ - 1)
                def _(k=k):
                    own_send_desc(0, k).start()

                @pl.when(z > 0)
                def _(k=k):
                    own_send_desc(1, k).start()

        issue_chunk(0, 0)
        for k in range(NQ):
            if k + 1 < NQ:
                issue_chunk(k + 1, (k + 1) % 2)
            q = quarter_of(k)
            wait_chunk(k, k % 2)
            part_ref[pl.ds(q * QR, QR), :] = stage_ref[k % 2].astype(jnp.bfloat16)
            ydesc(k).start()
            if k >= 1:
                combine(k - 1)
        combine(NQ - 1)

        def sidx(d, j):
            return (d - 1) * 2 + j

        def zwait_desc(side, d, j, o):
            rows = zsub_rows(o, x, j)
            return pltpu.make_async_remote_copy(
                src_ref=res_ref.at[rows],
                dst_ref=res_ref.at[rows],
                send_sem=zsend_sems.at[j],
                recv_sem=(zrecvL_sems if side == 0 else zrecvR_sems).at[sidx(d, j)],
                device_id=zleft if side == 0 else zright,
                device_id_type=pl.DeviceIdType.MESH,
            )

        def relay_desc(side, d, j, o):
            rows = zsub_rows(o, x, j)
            return pltpu.make_async_remote_copy(
                src_ref=res_ref.at[rows],
                dst_ref=res_ref.at[rows],
                send_sem=(relR_sems if side == 0 else relL_sems).at[sidx(d, j)],
                recv_sem=(zrecvL_sems if side == 0 else zrecvR_sems).at[
                    sidx(min(d + 1, ND), j)
                ],
                device_id=zright if side == 0 else zleft,
                device_id_type=pl.DeviceIdType.MESH,
            )

        def xfwd_desc(side, d, j, o):
            rows = zsub_rows(o, x, j)
            return pltpu.make_async_remote_copy(
                src_ref=res_ref.at[rows],
                dst_ref=res_ref.at[rows],
                send_sem=(xfwdL_sems if side == 0 else xfwdR_sems).at[sidx(d, j)],
                recv_sem=(xrecvL_sems if side == 0 else xrecvR_sems).at[sidx(d, j)],
                device_id=xpartner,
                device_id_type=pl.DeviceIdType.MESH,
            )

        def xwait_desc(side, d, j, o):
            rows = zsub_rows(o, 1 - x, j)
            return pltpu.make_async_remote_copy(
                src_ref=res_ref.at[rows],
                dst_ref=res_ref.at[rows],
                send_sem=(xfwdL_sems if side == 0 else xfwdR_sems).at[sidx(d, j)],
                recv_sem=(xrecvL_sems if side == 0 else xrecvR_sems).at[sidx(d, j)],
                device_id=xpartner,
                device_id_type=pl.DeviceIdType.MESH,
            )

        for d in range(1, NZ):
            for side in (0, 1):
                o = z - d if side == 0 else z + d
                valid = (o >= 0) if side == 0 else (o <= NZ - 1)

                @pl.when(valid)
                def _(side=side, d=d, o=o):
                    for j in range(2):
                        zwait_desc(side, d, j, o).wait_recv()
                        cont = (z < NZ - 1) if side == 0 else (z > 0)

                        @pl.when(cont)
                        def _(j=j):
                            relay_desc(side, d, j, o).start()

                        xfwd_desc(side, d, j, o).start()
                        rows = zsub_rows(o, x, j)
                        out_ref[rows, :] = res_ref[rows, :].astype(jnp.float32)

        for d in range(1, NZ):
            for side in (0, 1):
                o = z - d if side == 0 else z + d
                valid = (o >= 0) if side == 0 else (o <= NZ - 1)

                @pl.when(valid)
                def _(side=side, d=d, o=o):
                    for j in range(2):
                        xwait_desc(side, d, j, o).wait_recv()
                        rows = zsub_rows(o, 1 - x, j)
                        out_ref[rows, :] = res_ref[rows, :].astype(jnp.float32)

        for k in range(NQ):
            ydesc(k).wait_send()

        @pl.when(z < NZ - 1)
        def _():
            for j in range(2):
                own_send_desc(0, j).wait_send()

        @pl.when(z > 0)
        def _():
            for j in range(2):
                own_send_desc(1, j).wait_send()

        for d in range(1, NZ):
            for side in (0, 1):
                o = z - d if side == 0 else z + d
                valid = (o >= 0) if side == 0 else (o <= NZ - 1)

                @pl.when(valid)
                def _(side=side, d=d, o=o):
                    cont = (z < NZ - 1) if side == 0 else (z > 0)

                    @pl.when(cont)
                    def _():
                        for j in range(2):
                            relay_desc(side, d, j, o).wait_send()

                    for j in range(2):
                        xfwd_desc(side, d, j, o).wait_send()

    return pl.pallas_call(
        body,
        out_shape=jax.ShapeDtypeStruct((T, D), jnp.float32),
        in_specs=[
            pl.BlockSpec(memory_space=pltpu.SMEM),
            pl.BlockSpec(memory_space=pltpu.SMEM),
            pl.BlockSpec(memory_space=pltpu.VMEM),
            pl.BlockSpec(memory_space=pl.ANY),
        ],
        out_specs=pl.BlockSpec(memory_space=pltpu.VMEM),
        scratch_shapes=[
            pltpu.VMEM((S, D), jnp.bfloat16),
            pltpu.VMEM((S, D), jnp.bfloat16),
            pltpu.VMEM((T, D), jnp.bfloat16),
            pltpu.VMEM((2, QR, D), jnp.float32),
            pltpu.SemaphoreType.DMA((2,)),
            pltpu.SemaphoreType.DMA((NQ,)),
            pltpu.SemaphoreType.DMA((NQ,)),
            pltpu.SemaphoreType.DMA((4,)),
            pltpu.SemaphoreType.DMA((2 * ND,)),
            pltpu.SemaphoreType.DMA((2 * ND,)),
            pltpu.SemaphoreType.DMA((2 * ND,)),
            pltpu.SemaphoreType.DMA((2 * ND,)),
            pltpu.SemaphoreType.DMA((2 * ND,)),
            pltpu.SemaphoreType.DMA((2 * ND,)),
            pltpu.SemaphoreType.DMA((2 * ND,)),
            pltpu.SemaphoreType.DMA((2 * ND,)),
        ],
        compiler_params=pltpu.CompilerParams(collective_id=0),
    )(safe, inr, mask, E)


# device time: 45464 ns/iter; 2.1420x vs baseline; 1.0470x over previous
import jax
import jax.numpy as jnp
from jax import lax
from jax.experimental import pallas as pl
from jax.experimental.pallas import tpu as pltpu

T = 2048
V_LOCAL = 16384
D = 1024
NZ = 4
S = T // NZ
QR = 128
NQ = S // QR
ND = NZ - 1


def kernel(ids, E):
    my_y = lax.axis_index("y")
    offset = my_y * V_LOCAL
    local = ids - offset
    in_range = (local >= 0) & (local < V_LOCAL)
    safe = jnp.where(in_range, local, 0).astype(jnp.int32)
    inr = in_range.astype(jnp.int32)
    mask = in_range.astype(jnp.float32)[:, None]

    def body(ids_ref, mask_ref, e_ref, out_ref,
             part_ref, yrecv_ref, res_ref, stage_ref,
             gsems, ysend_sems, yrecv_sems,
             zsend_sems, zrecvL_sems, zrecvR_sems, relR_sems, relL_sems,
             xfwdL_sems, xfwdR_sems, xrecvL_sems, xrecvR_sems):
        x = lax.axis_index("x")
        y = lax.axis_index("y")
        z = lax.axis_index("z")
        ypartner = (x, 1 - y, z)
        xpartner = (1 - x, y, z)
        zleft = (x, y, z - 1)
        zright = (x, y, z + 1)
        base = z * S

        barrier = pltpu.get_barrier_semaphore()
        for peer in (ypartner, xpartner):
            pl.semaphore_signal(
                barrier, inc=1, device_id=peer,
                device_id_type=pl.DeviceIdType.MESH,
            )

        @pl.when(z > 0)
        def _():
            pl.semaphore_signal(
                barrier, inc=1, device_id=zleft,
                device_id_type=pl.DeviceIdType.MESH,
            )

        @pl.when(z < NZ - 1)
        def _():
            pl.semaphore_signal(
                barrier, inc=1, device_id=zright,
                device_id_type=pl.DeviceIdType.MESH,
            )

        nwait = 2 + (z > 0).astype(jnp.int32) + (z < NZ - 1).astype(jnp.int32)
        pl.semaphore_wait(barrier, nwait)

        def quarter_of(k):
            if k < 2:
                return x * 2 + k
            return (1 - x) * 2 + (k - 2)

        def issue_chunk(k, slot):
            gbase = base + quarter_of(k) * QR

            def issue(i, carry):
                pltpu.make_async_copy(
                    e_ref.at[ids_ref[gbase + i]],
                    stage_ref.at[slot, i],
                    gsems.at[slot],
                ).start()
                return carry

            lax.fori_loop(0, QR, issue, 0, unroll=16)

        def wait_chunk(k, slot):
            def w(i, carry):
                pltpu.make_async_copy(
                    e_ref.at[0],
                    stage_ref.at[slot, 0],
                    gsems.at[slot],
                ).wait()
                return carry

            lax.fori_loop(0, QR, w, 0, unroll=16)

        def ydesc(k):
            q = quarter_of(k)
            return pltpu.make_async_remote_copy(
                src_ref=part_ref.at[pl.ds(q * QR, QR)],
                dst_ref=yrecv_ref.at[pl.ds(q * QR, QR)],
                send_sem=ysend_sems.at[k],
                recv_sem=yrecv_sems.at[k],
                device_id=ypartner,
                device_id_type=pl.DeviceIdType.MESH,
            )

        def zsub_rows(origin, xcoord, j):
            return pl.ds(origin * S + xcoord * 2 * QR + j * QR, QR)

        def own_send_desc(direction, j):
            rows = zsub_rows(z, x, j)
            return pltpu.make_async_remote_copy(
                src_ref=res_ref.at[rows],
                dst_ref=res_ref.at[rows],
                send_sem=zsend_sems.at[direction * 2 + j],
                recv_sem=(zrecvL_sems if direction == 0 else zrecvR_sems).at[j],
                device_id=zright if direction == 0 else zleft,
                device_id_type=pl.DeviceIdType.MESH,
            )

        def combine(k):
            q = quarter_of(k)
            rows_g = pl.ds(base + q * QR, QR)
            ydesc(k).wait_recv()
            res_ref[rows_g, :] = jnp.where(
                mask_ref[rows_g, :] > 0.0,
                part_ref[pl.ds(q * QR, QR), :],
                yrecv_ref[pl.ds(q * QR, QR), :],
            )
            out_ref[rows_g, :] = res_ref[rows_g, :].astype(jnp.float32)
            if k < 2:
                @pl.when(z < NZ - 1)
                def _(k=k):
                    own_send_desc(0, k).start()

                @pl.when(z > 0)
                def _(k=k):
                    own_send_desc(1, k).start()

        issue_chunk(0, 0)
        for k in range(NQ):
            if k + 1 < NQ:
                issue_chunk(k + 1, (k + 1) % 2)
            q = quarter_of(k)
            wait_chunk(k, k % 2)
            part_ref[pl.ds(q * QR, QR), :] = stage_ref[k % 2].astype(jnp.bfloat16)
            ydesc(k).start()
            if k >= 1:
                combine(k - 1)
        combine(NQ - 1)

        def sidx(d, j):
            return (d - 1) * 2 + j

        def zwait_desc(side, d, j, o):
            rows = zsub_rows(o, x, j)
            return pltpu.make_async_remote_copy(
                src_ref=res_ref.at[rows],
                dst_ref=res_ref.at[rows],
                send_sem=zsend_sems.at[j],
                recv_sem=(zrecvL_sems if side == 0 else zrecvR_sems).at[sidx(d, j)],
                device_id=zleft if side == 0 else zright,
                device_id_type=pl.DeviceIdType.MESH,
            )

        def relay_desc(side, d, j, o):
            rows = zsub_rows(o, x, j)
            return pltpu.make_async_remote_copy(
                src_ref=res_ref.at[rows],
                dst_ref=res_ref.at[rows],
                send_sem=(relR_sems if side == 0 else relL_sems).at[sidx(d, j)],
                recv_sem=(zrecvL_sems if side == 0 else zrecvR_sems).at[
                    sidx(min(d + 1, ND), j)
                ],
                device_id=zright if side == 0 else zleft,
                device_id_type=pl.DeviceIdType.MESH,
            )

        def xfwd_desc(side, d, j, o):
            rows = zsub_rows(o, x, j)
            return pltpu.make_async_remote_copy(
                src_ref=res_ref.at[rows],
                dst_ref=res_ref.at[rows],
                send_sem=(xfwdL_sems if side == 0 else xfwdR_sems).at[sidx(d, j)],
                recv_sem=(xrecvL_sems if side == 0 else xrecvR_sems).at[sidx(d, j)],
                device_id=xpartner,
                device_id_type=pl.DeviceIdType.MESH,
            )

        def xwait_desc(side, d, j, o):
            rows = zsub_rows(o, 1 - x, j)
            return pltpu.make_async_remote_copy(
                src_ref=res_ref.at[rows],
                dst_ref=res_ref.at[rows],
                send_sem=(xfwdL_sems if side == 0 else xfwdR_sems).at[sidx(d, j)],
                recv_sem=(xrecvL_sems if side == 0 else xrecvR_sems).at[sidx(d, j)],
                device_id=xpartner,
                device_id_type=pl.DeviceIdType.MESH,
            )

        def zstep(d):
            for side in (0, 1):
                o = z - d if side == 0 else z + d
                valid = (o >= 0) if side == 0 else (o <= NZ - 1)

                @pl.when(valid)
                def _(side=side, d=d, o=o):
                    for j in range(2):
                        zwait_desc(side, d, j, o).wait_recv()
                        cont = (z < NZ - 1) if side == 0 else (z > 0)

                        @pl.when(cont)
                        def _(j=j):
                            relay_desc(side, d, j, o).start()

                        xfwd_desc(side, d, j, o).start()
                        rows = zsub_rows(o, x, j)
                        out_ref[rows, :] = res_ref[rows, :].astype(jnp.float32)

        def xstep(d):
            for side in (0, 1):
                o = z - d if side == 0 else z + d
                valid = (o >= 0) if side == 0 else (o <= NZ - 1)

                @pl.when(valid)
                def _(side=side, d=d, o=o):
                    for j in range(2):
                        xwait_desc(side, d, j, o).wait_recv()
                        rows = zsub_rows(o, 1 - x, j)
                        out_ref[rows, :] = res_ref[rows, :].astype(jnp.float32)

        zstep(1)
        zstep(2)
        xstep(1)
        zstep(3)
        xstep(2)
        xstep(3)

        for k in range(NQ):
            ydesc(k).wait_send()

        @pl.when(z < NZ - 1)
        def _():
            for j in range(2):
                own_send_desc(0, j).wait_send()

        @pl.when(z > 0)
        def _():
            for j in range(2):
                own_send_desc(1, j).wait_send()

        for d in range(1, NZ):
            for side in (0, 1):
                o = z - d if side == 0 else z + d
                valid = (o >= 0) if side == 0 else (o <= NZ - 1)

                @pl.when(valid)
                def _(side=side, d=d, o=o):
                    cont = (z < NZ - 1) if side == 0 else (z > 0)

                    @pl.when(cont)
                    def _():
                        for j in range(2):
                            relay_desc(side, d, j, o).wait_send()

                    for j in range(2):
                        xfwd_desc(side, d, j, o).wait_send()

    return pl.pallas_call(
        body,
        out_shape=jax.ShapeDtypeStruct((T, D), jnp.float32),
        in_specs=[
            pl.BlockSpec(memory_space=pltpu.SMEM),
            pl.BlockSpec(memory_space=pltpu.VMEM),
            pl.BlockSpec(memory_space=pl.ANY),
        ],
        out_specs=pl.BlockSpec(memory_space=pltpu.VMEM),
        scratch_shapes=[
            pltpu.VMEM((S, D), jnp.bfloat16),
            pltpu.VMEM((S, D), jnp.bfloat16),
            pltpu.VMEM((T, D), jnp.bfloat16),
            pltpu.VMEM((2, QR, D), jnp.float32),
            pltpu.SemaphoreType.DMA((2,)),
            pltpu.SemaphoreType.DMA((NQ,)),
            pltpu.SemaphoreType.DMA((NQ,)),
            pltpu.SemaphoreType.DMA((4,)),
            pltpu.SemaphoreType.DMA((2 * ND,)),
            pltpu.SemaphoreType.DMA((2 * ND,)),
            pltpu.SemaphoreType.DMA((2 * ND,)),
            pltpu.SemaphoreType.DMA((2 * ND,)),
            pltpu.SemaphoreType.DMA((2 * ND,)),
            pltpu.SemaphoreType.DMA((2 * ND,)),
            pltpu.SemaphoreType.DMA((2 * ND,)),
            pltpu.SemaphoreType.DMA((2 * ND,)),
        ],
        compiler_params=pltpu.CompilerParams(collective_id=0),
    )(safe, mask, E)


# device time: 43164 ns/iter; 2.2561x vs baseline; 1.0533x over previous
import jax
import jax.numpy as jnp
from jax import lax
from jax.experimental import pallas as pl
from jax.experimental.pallas import tpu as pltpu

T = 2048
V_LOCAL = 16384
D = 1024
NZ = 4
S = T // NZ
QR = 128
NQ = S // QR
ND = NZ - 1


def kernel(ids, E):
    my_y = lax.axis_index("y")
    offset = my_y * V_LOCAL
    local = ids - offset
    in_range = (local >= 0) & (local < V_LOCAL)
    safe = jnp.where(in_range, local, 0).astype(jnp.int32)
    inr = in_range.astype(jnp.int32)
    mask = in_range.astype(jnp.float32)[:, None]

    def body(ids_ref, mask_ref, e_ref, out_ref,
             part_ref, yrecv_ref, res_ref, stage_ref,
             gsems, ysend_sems, yrecv_sems, xown_send_sems, xown_recv_sems,
             zsend_sems, zrecvL_sems, zrecvR_sems, relR_sems, relL_sems,
             xfwdL_sems, xfwdR_sems, xrecvL_sems, xrecvR_sems):
        x = lax.axis_index("x")
        y = lax.axis_index("y")
        z = lax.axis_index("z")
        ypartner = (x, 1 - y, z)
        xpartner = (1 - x, y, z)
        zleft = (x, y, z - 1)
        zright = (x, y, z + 1)
        base = z * S

        barrier = pltpu.get_barrier_semaphore()
        for peer in (ypartner, xpartner):
            pl.semaphore_signal(
                barrier, inc=1, device_id=peer,
                device_id_type=pl.DeviceIdType.MESH,
            )

        @pl.when(z > 0)
        def _():
            pl.semaphore_signal(
                barrier, inc=1, device_id=zleft,
                device_id_type=pl.DeviceIdType.MESH,
            )

        @pl.when(z < NZ - 1)
        def _():
            pl.semaphore_signal(
                barrier, inc=1, device_id=zright,
                device_id_type=pl.DeviceIdType.MESH,
            )

        nwait = 2 + (z > 0).astype(jnp.int32) + (z < NZ - 1).astype(jnp.int32)
        pl.semaphore_wait(barrier, nwait)

        def quarter_of(k):
            return x * 2 + k

        def issue_chunk(k, slot):
            gbase = base + quarter_of(k) * QR

            def issue(i, carry):
                pltpu.make_async_copy(
                    e_ref.at[ids_ref[gbase + i]],
                    stage_ref.at[slot, i],
                    gsems.at[slot],
                ).start()
                return carry

            lax.fori_loop(0, QR, issue, 0, unroll=16)

        def wait_chunk(k, slot):
            def w(i, carry):
                pltpu.make_async_copy(
                    e_ref.at[0],
                    stage_ref.at[slot, 0],
                    gsems.at[slot],
                ).wait()
                return carry

            lax.fori_loop(0, QR, w, 0, unroll=16)

        def ydesc(k):
            return pltpu.make_async_remote_copy(
                src_ref=part_ref.at[pl.ds(k * QR, QR)],
                dst_ref=yrecv_ref.at[pl.ds(k * QR, QR)],
                send_sem=ysend_sems.at[k],
                recv_sem=yrecv_sems.at[k],
                device_id=ypartner,
                device_id_type=pl.DeviceIdType.MESH,
            )

        def zsub_rows(origin, xcoord, j):
            return pl.ds(origin * S + xcoord * 2 * QR + j * QR, QR)

        def own_send_desc(direction, j):
            rows = zsub_rows(z, x, j)
            return pltpu.make_async_remote_copy(
                src_ref=res_ref.at[rows],
                dst_ref=res_ref.at[rows],
                send_sem=zsend_sems.at[direction * 2 + j],
                recv_sem=(zrecvL_sems if direction == 0 else zrecvR_sems).at[j],
                device_id=zright if direction == 0 else zleft,
                device_id_type=pl.DeviceIdType.MESH,
            )

        def xown_desc(k):
            rows = zsub_rows(z, x, k)
            return pltpu.make_async_remote_copy(
                src_ref=res_ref.at[rows],
                dst_ref=res_ref.at[rows],
                send_sem=xown_send_sems.at[k],
                recv_sem=xown_recv_sems.at[k],
                device_id=xpartner,
                device_id_type=pl.DeviceIdType.MESH,
            )

        def xown_wait_desc(k):
            rows = zsub_rows(z, 1 - x, k)
            return pltpu.make_async_remote_copy(
                src_ref=res_ref.at[rows],
                dst_ref=res_ref.at[rows],
                send_sem=xown_send_sems.at[k],
                recv_sem=xown_recv_sems.at[k],
                device_id=xpartner,
                device_id_type=pl.DeviceIdType.MESH,
            )

        def combine(k):
            q = quarter_of(k)
            rows_g = pl.ds(base + q * QR, QR)
            ydesc(k).wait_recv()
            res_ref[rows_g, :] = jnp.where(
                mask_ref[rows_g, :] > 0.0,
                part_ref[pl.ds(k * QR, QR), :],
                yrecv_ref[pl.ds(k * QR, QR), :],
            )
            out_ref[rows_g, :] = res_ref[rows_g, :].astype(jnp.float32)
            @pl.when(z < NZ - 1)
            def _(k=k):
                own_send_desc(0, k).start()

            @pl.when(z > 0)
            def _(k=k):
                own_send_desc(1, k).start()

            xown_desc(k).start()

        issue_chunk(0, 0)
        for k in range(2):
            if k + 1 < 2:
                issue_chunk(k + 1, (k + 1) % 2)
            wait_chunk(k, k % 2)
            part_ref[pl.ds(k * QR, QR), :] = stage_ref[k % 2].astype(jnp.bfloat16)
            ydesc(k).start()
            if k >= 1:
                combine(k - 1)
        combine(1)

        def sidx(d, j):
            return (d - 1) * 2 + j

        def zwait_desc(side, d, j, o):
            rows = zsub_rows(o, x, j)
            return pltpu.make_async_remote_copy(
                src_ref=res_ref.at[rows],
                dst_ref=res_ref.at[rows],
                send_sem=zsend_sems.at[j],
                recv_sem=(zrecvL_sems if side == 0 else zrecvR_sems).at[sidx(d, j)],
                device_id=zleft if side == 0 else zright,
                device_id_type=pl.DeviceIdType.MESH,
            )

        def relay_desc(side, d, j, o):
            rows = zsub_rows(o, x, j)
            return pltpu.make_async_remote_copy(
                src_ref=res_ref.at[rows],
                dst_ref=res_ref.at[rows],
                send_sem=(relR_sems if side == 0 else relL_sems).at[sidx(d, j)],
                recv_sem=(zrecvL_sems if side == 0 else zrecvR_sems).at[
                    sidx(min(d + 1, ND), j)
                ],
                device_id=zright if side == 0 else zleft,
                device_id_type=pl.DeviceIdType.MESH,
            )

        def xfwd_desc(side, d, j, o):
            rows = zsub_rows(o, x, j)
            return pltpu.make_async_remote_copy(
                src_ref=res_ref.at[rows],
                dst_ref=res_ref.at[rows],
                send_sem=(xfwdL_sems if side == 0 else xfwdR_sems).at[sidx(d, j)],
                recv_sem=(xrecvL_sems if side == 0 else xrecvR_sems).at[sidx(d, j)],
                device_id=xpartner,
                device_id_type=pl.DeviceIdType.MESH,
            )

        def xwait_desc(side, d, j, o):
            rows = zsub_rows(o, 1 - x, j)
            return pltpu.make_async_remote_copy(
                src_ref=res_ref.at[rows],
                dst_ref=res_ref.at[rows],
                send_sem=(xfwdL_sems if side == 0 else xfwdR_sems).at[sidx(d, j)],
                recv_sem=(xrecvL_sems if side == 0 else xrecvR_sems).at[sidx(d, j)],
                device_id=xpartner,
                device_id_type=pl.DeviceIdType.MESH,
            )

        def zstep(d):
            for side in (0, 1):
                o = z - d if side == 0 else z + d
                valid = (o >= 0) if side == 0 else (o <= NZ - 1)

                @pl.when(valid)
                def _(side=side, d=d, o=o):
                    for j in range(2):
                        zwait_desc(side, d, j, o).wait_recv()
                        cont = (z < NZ - 1) if side == 0 else (z > 0)

                        @pl.when(cont)
                        def _(j=j):
                            relay_desc(side, d, j, o).start()

                        xfwd_desc(side, d, j, o).start()
                        rows = zsub_rows(o, x, j)
                        out_ref[rows, :] = res_ref[rows, :].astype(jnp.float32)

        def xstep(d):
            for side in (0, 1):
                o = z - d if side == 0 else z + d
                valid = (o >= 0) if side == 0 else (o <= NZ - 1)

                @pl.when(valid)
                def _(side=side, d=d, o=o):
                    for j in range(2):
                        xwait_desc(side, d, j, o).wait_recv()
                        rows = zsub_rows(o, 1 - x, j)
                        out_ref[rows, :] = res_ref[rows, :].astype(jnp.float32)

        zstep(1)
        for k in range(2):
            xown_wait_desc(k).wait_recv()
            rows = zsub_rows(z, 1 - x, k)
            out_ref[rows, :] = res_ref[rows, :].astype(jnp.float32)
        zstep(2)
        xstep(1)
        zstep(3)
        xstep(2)
        xstep(3)

        for k in range(2):
            ydesc(k).wait_send()
            xown_desc(k).wait_send()

        @pl.when(z < NZ - 1)
        def _():
            for j in range(2):
                own_send_desc(0, j).wait_send()

        @pl.when(z > 0)
        def _():
            for j in range(2):
                own_send_desc(1, j).wait_send()

        for d in range(1, NZ):
            for side in (0, 1):
                o = z - d if side == 0 else z + d
                valid = (o >= 0) if side == 0 else (o <= NZ - 1)

                @pl.when(valid)
                def _(side=side, d=d, o=o):
                    cont = (z < NZ - 1) if side == 0 else (z > 0)

                    @pl.when(cont)
                    def _():
                        for j in range(2):
                            relay_desc(side, d, j, o).wait_send()

                    for j in range(2):
                        xfwd_desc(side, d, j, o).wait_send()

    return pl.pallas_call(
        body,
        out_shape=jax.ShapeDtypeStruct((T, D), jnp.float32),
        in_specs=[
            pl.BlockSpec(memory_space=pltpu.SMEM),
            pl.BlockSpec(memory_space=pltpu.VMEM),
            pl.BlockSpec(memory_space=pl.ANY),
        ],
        out_specs=pl.BlockSpec(memory_space=pltpu.VMEM),
        scratch_shapes=[
            pltpu.VMEM((2 * QR, D), jnp.bfloat16),
            pltpu.VMEM((2 * QR, D), jnp.bfloat16),
            pltpu.VMEM((T, D), jnp.bfloat16),
            pltpu.VMEM((2, QR, D), jnp.float32),
            pltpu.SemaphoreType.DMA((2,)),
            pltpu.SemaphoreType.DMA((2,)),
            pltpu.SemaphoreType.DMA((2,)),
            pltpu.SemaphoreType.DMA((2,)),
            pltpu.SemaphoreType.DMA((2,)),
            pltpu.SemaphoreType.DMA((4,)),
            pltpu.SemaphoreType.DMA((2 * ND,)),
            pltpu.SemaphoreType.DMA((2 * ND,)),
            pltpu.SemaphoreType.DMA((2 * ND,)),
            pltpu.SemaphoreType.DMA((2 * ND,)),
            pltpu.SemaphoreType.DMA((2 * ND,)),
            pltpu.SemaphoreType.DMA((2 * ND,)),
            pltpu.SemaphoreType.DMA((2 * ND,)),
            pltpu.SemaphoreType.DMA((2 * ND,)),
        ],
        compiler_params=pltpu.CompilerParams(collective_id=0),
    )(safe, mask, E)


# device time: 41823 ns/iter; 2.3284x vs baseline; 1.0321x over previous
import jax
import jax.numpy as jnp
from jax import lax
from jax.experimental import pallas as pl
from jax.experimental.pallas import tpu as pltpu

T = 2048
V_LOCAL = 16384
D = 1024
NZ = 4
S = T // NZ
QR = 128
NQ = S // QR
ND = NZ - 1


def kernel(ids, E):
    my_y = lax.axis_index("y")
    offset = my_y * V_LOCAL
    local = ids - offset
    in_range = (local >= 0) & (local < V_LOCAL)
    safe = jnp.where(in_range, local, 0).astype(jnp.int32)
    inr = in_range.astype(jnp.int32)
    mask = in_range.astype(jnp.float32)[:, None]

    def body(ids_ref, mask_ref, e_ref, out_ref,
             part_ref, yrecv_ref, stage_ref,
             gsems, ysend_sems, yrecv_sems, xown_send_sems, xown_recv_sems,
             zsend_sems, zrecvL_sems, zrecvR_sems, relR_sems, relL_sems,
             xfwdL_sems, xfwdR_sems, xrecvL_sems, xrecvR_sems):
        x = lax.axis_index("x")
        y = lax.axis_index("y")
        z = lax.axis_index("z")
        ypartner = (x, 1 - y, z)
        xpartner = (1 - x, y, z)
        zleft = (x, y, z - 1)
        zright = (x, y, z + 1)
        base = z * S

        barrier = pltpu.get_barrier_semaphore()
        for peer in (ypartner, xpartner):
            pl.semaphore_signal(
                barrier, inc=1, device_id=peer,
                device_id_type=pl.DeviceIdType.MESH,
            )

        @pl.when(z > 0)
        def _():
            pl.semaphore_signal(
                barrier, inc=1, device_id=zleft,
                device_id_type=pl.DeviceIdType.MESH,
            )

        @pl.when(z < NZ - 1)
        def _():
            pl.semaphore_signal(
                barrier, inc=1, device_id=zright,
                device_id_type=pl.DeviceIdType.MESH,
            )

        nwait = 2 + (z > 0).astype(jnp.int32) + (z < NZ - 1).astype(jnp.int32)
        pl.semaphore_wait(barrier, nwait)

        def quarter_of(k):
            return x * 2 + k

        def issue_chunk(k, slot):
            gbase = base + quarter_of(k) * QR

            def issue(i, carry):
                pltpu.make_async_copy(
                    e_ref.at[ids_ref[gbase + i]],
                    stage_ref.at[slot, i],
                    gsems.at[slot],
                ).start()
                return carry

            lax.fori_loop(0, QR, issue, 0, unroll=16)

        def wait_chunk(k, slot):
            def w(i, carry):
                pltpu.make_async_copy(
                    e_ref.at[0],
                    stage_ref.at[slot, 0],
                    gsems.at[slot],
                ).wait()
                return carry

            lax.fori_loop(0, QR, w, 0, unroll=16)

        def ydesc(k):
            return pltpu.make_async_remote_copy(
                src_ref=part_ref.at[pl.ds(k * QR, QR)],
                dst_ref=yrecv_ref.at[pl.ds(k * QR, QR)],
                send_sem=ysend_sems.at[k],
                recv_sem=yrecv_sems.at[k],
                device_id=ypartner,
                device_id_type=pl.DeviceIdType.MESH,
            )

        def zsub_rows(origin, xcoord, j):
            return pl.ds(origin * S + xcoord * 2 * QR + j * QR, QR)

        def own_send_desc(direction, j):
            rows = zsub_rows(z, x, j)
            return pltpu.make_async_remote_copy(
                src_ref=out_ref.at[rows],
                dst_ref=out_ref.at[rows],
                send_sem=zsend_sems.at[direction * 2 + j],
                recv_sem=(zrecvL_sems if direction == 0 else zrecvR_sems).at[j],
                device_id=zright if direction == 0 else zleft,
                device_id_type=pl.DeviceIdType.MESH,
            )

        def xown_desc(k):
            rows = zsub_rows(z, x, k)
            return pltpu.make_async_remote_copy(
                src_ref=out_ref.at[rows],
                dst_ref=out_ref.at[rows],
                send_sem=xown_send_sems.at[k],
                recv_sem=xown_recv_sems.at[k],
                device_id=xpartner,
                device_id_type=pl.DeviceIdType.MESH,
            )

        def xown_wait_desc(k):
            rows = zsub_rows(z, 1 - x, k)
            return pltpu.make_async_remote_copy(
                src_ref=out_ref.at[rows],
                dst_ref=out_ref.at[rows],
                send_sem=xown_send_sems.at[k],
                recv_sem=xown_recv_sems.at[k],
                device_id=xpartner,
                device_id_type=pl.DeviceIdType.MESH,
            )

        def combine(k):
            q = quarter_of(k)
            rows_g = pl.ds(base + q * QR, QR)
            ydesc(k).wait_recv()
            out_ref[rows_g, :] = jnp.where(
                mask_ref[rows_g, :] > 0.0,
                part_ref[pl.ds(k * QR, QR), :],
                yrecv_ref[pl.ds(k * QR, QR), :],
            )
            @pl.when(z < NZ - 1)
            def _(k=k):
                own_send_desc(0, k).start()

            @pl.when(z > 0)
            def _(k=k):
                own_send_desc(1, k).start()

            xown_desc(k).start()

        issue_chunk(0, 0)
        for k in range(2):
            wait_chunk(k, k % 2)
            part_ref[pl.ds(k * QR, QR), :] = stage_ref[k % 2].astype(jnp.bfloat16)
            ydesc(k).start()
            if k + 1 < 2:
                issue_chunk(k + 1, (k + 1) % 2)
            if k >= 1:
                combine(k - 1)
        combine(1)

        def sidx(d, j):
            return (d - 1) * 2 + j

        def zwait_desc(side, d, j, o):
            rows = zsub_rows(o, x, j)
            return pltpu.make_async_remote_copy(
                src_ref=out_ref.at[rows],
                dst_ref=out_ref.at[rows],
                send_sem=zsend_sems.at[j],
                recv_sem=(zrecvL_sems if side == 0 else zrecvR_sems).at[sidx(d, j)],
                device_id=zleft if side == 0 else zright,
                device_id_type=pl.DeviceIdType.MESH,
            )

        def relay_desc(side, d, j, o):
            rows = zsub_rows(o, x, j)
            return pltpu.make_async_remote_copy(
                src_ref=out_ref.at[rows],
                dst_ref=out_ref.at[rows],
                send_sem=(relR_sems if side == 0 else relL_sems).at[sidx(d, j)],
                recv_sem=(zrecvL_sems if side == 0 else zrecvR_sems).at[
                    sidx(min(d + 1, ND), j)
                ],
                device_id=zright if side == 0 else zleft,
                device_id_type=pl.DeviceIdType.MESH,
            )

        def xfwd_desc(side, d, j, o):
            rows = zsub_rows(o, x, j)
            return pltpu.make_async_remote_copy(
                src_ref=out_ref.at[rows],
                dst_ref=out_ref.at[rows],
                send_sem=(xfwdL_sems if side == 0 else xfwdR_sems).at[sidx(d, j)],
                recv_sem=(xrecvL_sems if side == 0 else xrecvR_sems).at[sidx(d, j)],
                device_id=xpartner,
                device_id_type=pl.DeviceIdType.MESH,
            )

        def xwait_desc(side, d, j, o):
            rows = zsub_rows(o, 1 - x, j)
            return pltpu.make_async_remote_copy(
                src_ref=out_ref.at[rows],
                dst_ref=out_ref.at[rows],
                send_sem=(xfwdL_sems if side == 0 else xfwdR_sems).at[sidx(d, j)],
                recv_sem=(xrecvL_sems if side == 0 else xrecvR_sems).at[sidx(d, j)],
                device_id=xpartner,
                device_id_type=pl.DeviceIdType.MESH,
            )

        def zstep(d):
            for side in (0, 1):
                o = z - d if side == 0 else z + d
                valid = (o >= 0) if side == 0 else (o <= NZ - 1)

                @pl.when(valid)
                def _(side=side, d=d, o=o):
                    for j in range(2):
                        zwait_desc(side, d, j, o).wait_recv()
                        cont = (z < NZ - 1) if side == 0 else (z > 0)

                        @pl.when(cont)
                        def _(j=j):
                            relay_desc(side, d, j, o).start()

                        xfwd_desc(side, d, j, o).start()

        def xstep(d):
            for side in (0, 1):
                o = z - d if side == 0 else z + d
                valid = (o >= 0) if side == 0 else (o <= NZ - 1)

                @pl.when(valid)
                def _(side=side, d=d, o=o):
                    for j in range(2):
                        xwait_desc(side, d, j, o).wait_recv()

        zstep(1)
        for k in range(2):
            xown_wait_desc(k).wait_recv()
        zstep(2)
        xstep(1)
        zstep(3)
        xstep(2)
        xstep(3)

        for k in range(2):
            ydesc(k).wait_send()
            xown_desc(k).wait_send()

        @pl.when(z < NZ - 1)
        def _():
            for j in range(2):
                own_send_desc(0, j).wait_send()

        @pl.when(z > 0)
        def _():
            for j in range(2):
                own_send_desc(1, j).wait_send()

        for d in range(1, NZ):
            for side in (0, 1):
                o = z - d if side == 0 else z + d
                valid = (o >= 0) if side == 0 else (o <= NZ - 1)

                @pl.when(valid)
                def _(side=side, d=d, o=o):
                    cont = (z < NZ - 1) if side == 0 else (z > 0)

                    @pl.when(cont)
                    def _():
                        for j in range(2):
                            relay_desc(side, d, j, o).wait_send()

                    for j in range(2):
                        xfwd_desc(side, d, j, o).wait_send()

    return pl.pallas_call(
        body,
        out_shape=jax.ShapeDtypeStruct((T, D), jnp.bfloat16),
        in_specs=[
            pl.BlockSpec(memory_space=pltpu.SMEM),
            pl.BlockSpec(memory_space=pltpu.VMEM),
            pl.BlockSpec(memory_space=pl.ANY),
        ],
        out_specs=pl.BlockSpec(memory_space=pltpu.VMEM),
        scratch_shapes=[
            pltpu.VMEM((2 * QR, D), jnp.bfloat16),
            pltpu.VMEM((2 * QR, D), jnp.bfloat16),
            pltpu.VMEM((2, QR, D), jnp.float32),
            pltpu.SemaphoreType.DMA((2,)),
            pltpu.SemaphoreType.DMA((2,)),
            pltpu.SemaphoreType.DMA((2,)),
            pltpu.SemaphoreType.DMA((2,)),
            pltpu.SemaphoreType.DMA((2,)),
            pltpu.SemaphoreType.DMA((4,)),
            pltpu.SemaphoreType.DMA((2 * ND,)),
            pltpu.SemaphoreType.DMA((2 * ND,)),
            pltpu.SemaphoreType.DMA((2 * ND,)),
            pltpu.SemaphoreType.DMA((2 * ND,)),
            pltpu.SemaphoreType.DMA((2 * ND,)),
            pltpu.SemaphoreType.DMA((2 * ND,)),
            pltpu.SemaphoreType.DMA((2 * ND,)),
            pltpu.SemaphoreType.DMA((2 * ND,)),
        ],
        compiler_params=pltpu.CompilerParams(collective_id=0),
    )(safe, mask, E)


# device time: 41803 ns/iter; 2.3295x vs baseline; 1.0005x over previous
import jax
import jax.numpy as jnp
from jax import lax
from jax.experimental import pallas as pl
from jax.experimental.pallas import tpu as pltpu

T = 2048
V_LOCAL = 16384
D = 1024
NZ = 4
S = T // NZ
QR = 128
ND = NZ - 1


def kernel(ids, E):
    my_y = lax.axis_index("y")
    offset = my_y * V_LOCAL
    local = ids - offset
    in_range = (local >= 0) & (local < V_LOCAL)
    safe = jnp.where(in_range, local, 0).astype(jnp.int32)
    mask = in_range.astype(jnp.float32)[:, None]

    def body(ids_ref, mask_ref, e_ref, out_ref,
             part_ref, yrecv_ref, stage_ref,
             gsems, ysend_sems, yrecv_sems, xown_send_sems, xown_recv_sems,
             zsend_sems, zrecvL_sems, zrecvR_sems, relR_sems, relL_sems,
             xfwdL_sems, xfwdR_sems, xrecvL_sems, xrecvR_sems):
        x = lax.axis_index("x")
        y = lax.axis_index("y")
        z = lax.axis_index("z")
        ypartner = (x, 1 - y, z)
        xpartner = (1 - x, y, z)
        zleft = (x, y, z - 1)
        zright = (x, y, z + 1)
        base = z * S

        barrier = pltpu.get_barrier_semaphore()
        for peer in (ypartner, xpartner):
            pl.semaphore_signal(
                barrier, inc=1, device_id=peer,
                device_id_type=pl.DeviceIdType.MESH,
            )

        @pl.when(z > 0)
        def _():
            pl.semaphore_signal(
                barrier, inc=1, device_id=zleft,
                device_id_type=pl.DeviceIdType.MESH,
            )

        @pl.when(z < NZ - 1)
        def _():
            pl.semaphore_signal(
                barrier, inc=1, device_id=zright,
                device_id_type=pl.DeviceIdType.MESH,
            )

        nwait = 2 + (z > 0).astype(jnp.int32) + (z < NZ - 1).astype(jnp.int32)
        pl.semaphore_wait(barrier, nwait)

        def quarter_of(k):
            return x * 2 + k

        def issue_chunk(k, slot):
            gbase = base + quarter_of(k) * QR

            def issue(i, carry):
                pltpu.make_async_copy(
                    e_ref.at[ids_ref[gbase + i]],
                    stage_ref.at[slot, i],
                    gsems.at[slot],
                ).start()
                return carry

            lax.fori_loop(0, QR, issue, 0, unroll=16)

        def wait_chunk(k, slot):
            def w(i, carry):
                pltpu.make_async_copy(
                    e_ref.at[0],
                    stage_ref.at[slot, 0],
                    gsems.at[slot],
                ).wait()
                return carry

            lax.fori_loop(0, QR, w, 0, unroll=16)

        def ydesc(k):
            return pltpu.make_async_remote_copy(
                src_ref=part_ref.at[pl.ds(k * QR, QR)],
                dst_ref=yrecv_ref.at[pl.ds(k * QR, QR)],
                send_sem=ysend_sems.at[k],
                recv_sem=yrecv_sems.at[k],
                device_id=ypartner,
                device_id_type=pl.DeviceIdType.MESH,
            )

        def zsub_rows(origin, xcoord, j):
            return pl.ds(origin * S + xcoord * 2 * QR + j * QR, QR)

        def own_send_desc(direction, j):
            rows = zsub_rows(z, x, j)
            return pltpu.make_async_remote_copy(
                src_ref=out_ref.at[rows],
                dst_ref=out_ref.at[rows],
                send_sem=zsend_sems.at[direction * 2 + j],
                recv_sem=(zrecvL_sems if direction == 0 else zrecvR_sems).at[j],
                device_id=zright if direction == 0 else zleft,
                device_id_type=pl.DeviceIdType.MESH,
            )

        def xown_desc(k):
            rows = zsub_rows(z, x, k)
            return pltpu.make_async_remote_copy(
                src_ref=out_ref.at[rows],
                dst_ref=out_ref.at[rows],
                send_sem=xown_send_sems.at[k],
                recv_sem=xown_recv_sems.at[k],
                device_id=xpartner,
                device_id_type=pl.DeviceIdType.MESH,
            )

        def xown_wait_desc(k):
            rows = zsub_rows(z, 1 - x, k)
            return pltpu.make_async_remote_copy(
                src_ref=out_ref.at[rows],
                dst_ref=out_ref.at[rows],
                send_sem=xown_send_sems.at[k],
                recv_sem=xown_recv_sems.at[k],
                device_id=xpartner,
                device_id_type=pl.DeviceIdType.MESH,
            )

        def combine(k):
            q = quarter_of(k)
            rows_g = pl.ds(base + q * QR, QR)
            ydesc(k).wait_recv()
            out_ref[rows_g, :] = jnp.where(
                mask_ref[rows_g, :] > 0.0,
                part_ref[pl.ds(k * QR, QR), :],
                yrecv_ref[pl.ds(k * QR, QR), :],
            )
            @pl.when(z < NZ - 1)
            def _(k=k):
                own_send_desc(0, k).start()

            @pl.when(z > 0)
            def _(k=k):
                own_send_desc(1, k).start()

            xown_desc(k).start()

        issue_chunk(0, 0)
        for k in range(2):
            wait_chunk(k, k % 2)
            part_ref[pl.ds(k * QR, QR), :] = stage_ref[k % 2].astype(jnp.bfloat16)
            ydesc(k).start()
            if k + 1 < 2:
                issue_chunk(k + 1, (k + 1) % 2)
            if k >= 1:
                combine(k - 1)
        combine(1)

        def sidx(d, j):
            return (d - 1) * 2 + j

        def zwait_desc(side, d, j, o):
            rows = zsub_rows(o, x, j)
            return pltpu.make_async_remote_copy(
                src_ref=out_ref.at[rows],
                dst_ref=out_ref.at[rows],
                send_sem=zsend_sems.at[j],
                recv_sem=(zrecvL_sems if side == 0 else zrecvR_sems).at[sidx(d, j)],
                device_id=zleft if side == 0 else zright,
                device_id_type=pl.DeviceIdType.MESH,
            )

        def relay_desc(side, d, j, o):
            rows = zsub_rows(o, x, j)
            return pltpu.make_async_remote_copy(
                src_ref=out_ref.at[rows],
                dst_ref=out_ref.at[rows],
                send_sem=(relR_sems if side == 0 else relL_sems).at[sidx(d, j)],
                recv_sem=(zrecvL_sems if side == 0 else zrecvR_sems).at[
                    sidx(min(d + 1, ND), j)
                ],
                device_id=zright if side == 0 else zleft,
                device_id_type=pl.DeviceIdType.MESH,
            )

        def xfwd_desc(side, d, j, o):
            rows = zsub_rows(o, x, j)
            return pltpu.make_async_remote_copy(
                src_ref=out_ref.at[rows],
                dst_ref=out_ref.at[rows],
                send_sem=(xfwdL_sems if side == 0 else xfwdR_sems).at[sidx(d, j)],
                recv_sem=(xrecvL_sems if side == 0 else xrecvR_sems).at[sidx(d, j)],
                device_id=xpartner,
                device_id_type=pl.DeviceIdType.MESH,
            )

        def xwait_desc(side, d, j, o):
            rows = zsub_rows(o, 1 - x, j)
            return pltpu.make_async_remote_copy(
                src_ref=out_ref.at[rows],
                dst_ref=out_ref.at[rows],
                send_sem=(xfwdL_sems if side == 0 else xfwdR_sems).at[sidx(d, j)],
                recv_sem=(xrecvL_sems if side == 0 else xrecvR_sems).at[sidx(d, j)],
                device_id=xpartner,
                device_id_type=pl.DeviceIdType.MESH,
            )

        def zstep(d):
            for side in (0, 1):
                o = z - d if side == 0 else z + d
                valid = (o >= 0) if side == 0 else (o <= NZ - 1)

                @pl.when(valid)
                def _(side=side, d=d, o=o):
                    for j in range(2):
                        zwait_desc(side, d, j, o).wait_recv()
                        cont = (z < NZ - 1) if side == 0 else (z > 0)

                        @pl.when(cont)
                        def _(j=j):
                            relay_desc(side, d, j, o).start()

                        xfwd_desc(side, d, j, o).start()

        def xstep(d):
            for side in (0, 1):
                o = z - d if side == 0 else z + d
                valid = (o >= 0) if side == 0 else (o <= NZ - 1)

                @pl.when(valid)
                def _(side=side, d=d, o=o):
                    for j in range(2):
                        xwait_desc(side, d, j, o).wait_recv()

        zstep(1)
        for k in range(2):
            xown_wait_desc(k).wait_recv()
        zstep(2)
        xstep(1)
        zstep(3)
        xstep(2)
        xstep(3)

        for k in range(2):
            ydesc(k).wait_send()
            xown_desc(k).wait_send()

        @pl.when(z < NZ - 1)
        def _():
            for j in range(2):
                own_send_desc(0, j).wait_send()

        @pl.when(z > 0)
        def _():
            for j in range(2):
                own_send_desc(1, j).wait_send()

        for d in range(1, NZ):
            for side in (0, 1):
                o = z - d if side == 0 else z + d
                valid = (o >= 0) if side == 0 else (o <= NZ - 1)

                @pl.when(valid)
                def _(side=side, d=d, o=o):
                    cont = (z < NZ - 1) if side == 0 else (z > 0)

                    @pl.when(cont)
                    def _():
                        for j in range(2):
                            relay_desc(side, d, j, o).wait_send()

                    for j in range(2):
                        xfwd_desc(side, d, j, o).wait_send()

    return pl.pallas_call(
        body,
        out_shape=jax.ShapeDtypeStruct((T, D), jnp.bfloat16),
        in_specs=[
            pl.BlockSpec(memory_space=pltpu.SMEM),
            pl.BlockSpec(memory_space=pltpu.VMEM),
            pl.BlockSpec(memory_space=pl.ANY),
        ],
        out_specs=pl.BlockSpec(memory_space=pltpu.VMEM),
        scratch_shapes=[
            pltpu.VMEM((2 * QR, D), jnp.bfloat16),
            pltpu.VMEM((2 * QR, D), jnp.bfloat16),
            pltpu.VMEM((2, QR, D), jnp.float32),
            pltpu.SemaphoreType.DMA((2,)),
            pltpu.SemaphoreType.DMA((2,)),
            pltpu.SemaphoreType.DMA((2,)),
            pltpu.SemaphoreType.DMA((2,)),
            pltpu.SemaphoreType.DMA((2,)),
            pltpu.SemaphoreType.DMA((4,)),
            pltpu.SemaphoreType.DMA((2 * ND,)),
            pltpu.SemaphoreType.DMA((2 * ND,)),
            pltpu.SemaphoreType.DMA((2 * ND,)),
            pltpu.SemaphoreType.DMA((2 * ND,)),
            pltpu.SemaphoreType.DMA((2 * ND,)),
            pltpu.SemaphoreType.DMA((2 * ND,)),
            pltpu.SemaphoreType.DMA((2 * ND,)),
            pltpu.SemaphoreType.DMA((2 * ND,)),
        ],
        compiler_params=pltpu.CompilerParams(collective_id=0),
    )(safe, mask, E)
